# baseline (speedup 1.0000x reference)
import functools

import jax
import jax.numpy as jnp
from jax import lax
from jax.experimental import pallas as pl
from jax.experimental.pallas import tpu as pltpu
from jax.experimental.pallas import tpu_sc as plsc

EMBED_DIM = 32
ROW_W = 128
RPG = ROW_W // EMBED_DIM
CHUNK = 128

_info = plsc.get_sparse_core_info()
_NC, _NS = _info.num_cores, _info.num_subcores
_NW = _NC * _NS


@functools.partial(jax.jit, static_argnums=(2,))
def _gather(table4, idx, rows_per_w):
    n = idx.shape[0]
    n_chunks = rows_per_w // CHUNK
    mesh = plsc.VectorSubcoreMesh(core_axis_name="c", subcore_axis_name="s")

    @functools.partial(
        pl.kernel,
        mesh=mesh,
        out_type=jax.ShapeDtypeStruct((n, ROW_W), jnp.float32),
        scratch_types=[
            pltpu.VMEM((rows_per_w,), jnp.int32),
            pltpu.VMEM((CHUNK, ROW_W), jnp.float32),
            pltpu.VMEM((CHUNK, ROW_W), jnp.float32),
            pltpu.SemaphoreType.DMA,
        ],
    )
    def k(table_hbm, idx_hbm, out_hbm, idx_v, rows_a, rows_b, gsem):
        wid = lax.axis_index("s") * _NC + lax.axis_index("c")
        base = wid * rows_per_w
        pltpu.sync_copy(idx_hbm.at[pl.ds(base, rows_per_w)], idx_v)

        def fire(c, buf):
            for s in range(CHUNK // 16):
                qi = idx_v[pl.ds(c * CHUNK + s * 16, 16)] >> 2
                pltpu.async_copy(table_hbm.at[qi], buf.at[pl.ds(s * 16, 16)], gsem)

        def drain(c, buf):
            for s in range(CHUNK // 16):
                qi = idx_v[pl.ds(c * CHUNK + s * 16, 16)] >> 2
                pltpu.make_async_copy(
                    table_hbm.at[qi], buf.at[pl.ds(s * 16, 16)], gsem
                ).wait()

        def write(c, buf):
            pltpu.sync_copy(buf, out_hbm.at[pl.ds(base + c * CHUNK, CHUNK)])

        fire(0, rows_a)

        def step(g, carry):
            c = g * 2
            drain(c, rows_a)
            fire(c + 1, rows_b)
            write(c, rows_a)
            drain(c + 1, rows_b)
            fire(c + 2, rows_a)
            write(c + 1, rows_b)
            return carry

        lax.fori_loop(0, n_chunks // 2 - 1, step, 0)
        c = n_chunks - 2
        drain(c, rows_a)
        fire(c + 1, rows_b)
        write(c, rows_a)
        drain(c + 1, rows_b)
        write(c + 1, rows_b)

    return k(table4, idx)


def kernel(spatialgrid, comparison_grid):
    batch, n_fields = comparison_grid.shape[0], comparison_grid.shape[1]
    n = batch * n_fields
    idx = comparison_grid.reshape(n)
    table4 = spatialgrid.reshape(spatialgrid.shape[0] // RPG, ROW_W)
    rows4 = _gather(table4, idx, n // _NW)
    m = (idx & 3).reshape(n, 1)
    out = jnp.zeros((n, EMBED_DIM), jnp.float32)
    for sub in range(RPG):
        sel = (m == sub).astype(jnp.float32)
        out = out + sel * rows4[:, sub * EMBED_DIM : (sub + 1) * EMBED_DIM]
    return out.reshape(batch, n_fields, EMBED_DIM)


# speedup vs baseline: 1.3024x; 1.3024x over previous
import functools

import jax
import jax.numpy as jnp
from jax import lax
from jax.experimental import pallas as pl
from jax.experimental.pallas import tpu as pltpu
from jax.experimental.pallas import tpu_sc as plsc

EMBED_DIM = 32

_info = plsc.get_sparse_core_info()
_NC, _NS = _info.num_cores, _info.num_subcores
_NW = _NC * _NS


@functools.partial(jax.jit, static_argnums=(2,))
def _gather(table, idx, b_per_w):
    mesh = plsc.VectorSubcoreMesh(core_axis_name="c", subcore_axis_name="s")

    @functools.partial(
        pl.kernel,
        mesh=mesh,
        compiler_params=pltpu.CompilerParams(use_tc_tiling_on_sc=False),
        out_type=jax.ShapeDtypeStruct((b_per_w * _NW, EMBED_DIM), jnp.float32),
        scratch_types=[
            pltpu.VMEM((b_per_w,), jnp.int32),
            pltpu.VMEM((b_per_w, EMBED_DIM), jnp.float32),
            pltpu.SemaphoreType.DMA,
        ],
    )
    def k(table_hbm, idx_hbm, out_hbm, idx_v, rows_v, sem):
        wid = lax.axis_index("s") * _NC + lax.axis_index("c")
        base = wid * b_per_w
        pltpu.sync_copy(idx_hbm.at[pl.ds(base, b_per_w)], idx_v)
        pltpu.async_copy(table_hbm.at[idx_v], rows_v, sem).wait()
        pltpu.sync_copy(rows_v, out_hbm.at[pl.ds(base, b_per_w)])

    return k(table, idx)


def kernel(spatialgrid, comparison_grid):
    batch, n_fields = comparison_grid.shape[0], comparison_grid.shape[1]
    n = batch * n_fields
    b_per_w = n // _NW
    idx = comparison_grid.reshape(n)
    out = _gather(spatialgrid, idx, b_per_w)
    return out.reshape(batch, n_fields, EMBED_DIM)
